# baseline (speedup 1.0000x reference)
import jax
import jax.numpy as jnp
from jax import lax
from jax.experimental import pallas as pl
from jax.experimental.pallas import tpu as pltpu
from jax.experimental.pallas import tpu_sc as plsc

L = 16
R, C = 64, 32768
NC, NS = 2, 16
NW = NC * NS
RPW = R // NW
CHUNKS = C // L
ACC = 4
UNR = 8
SPR = 2
SEG = C // SPR
NSEG = RPW * SPR

_NEG = float("-inf")


def _insert(t0, t1, t2, v):
    lo = jnp.minimum(t0, v)
    t0 = jnp.maximum(t0, v)
    lo2 = jnp.minimum(t1, lo)
    t1 = jnp.maximum(t1, lo)
    t2 = jnp.maximum(t2, lo2)
    return t0, t1, t2


def _tec_body(x_hbm, out_hbm, xv, resv, s0, s1, s2, s3):
    cid = lax.axis_index("c")
    sid = lax.axis_index("s")
    wid = sid * NC + cid
    base = wid * RPW
    sems = [s0, s1, s2, s3]
    copies = []
    for k in range(NSEG):
        copies.append(pltpu.async_copy(
            x_hbm.at[pl.ds(base * C + k * SEG, SEG)],
            xv.at[pl.ds(k * SEG, SEG)],
            sems[k]))
    lane = lax.iota(jnp.int32, L)
    full = jnp.full((L,), _NEG, jnp.float32)
    for r in range(RPW):
        ts = (full,) * (3 * ACC)
        for s in range(SPR):
            copies[r * SPR + s].wait()

            def step(i, carry, _off=r * C + s * SEG):
                acc = list(carry)
                off = _off + i * (L * UNR)
                for j in range(UNR):
                    a = j % ACC
                    v = xv[pl.ds(off + j * L, L)]
                    acc[3 * a], acc[3 * a + 1], acc[3 * a + 2] = _insert(
                        acc[3 * a], acc[3 * a + 1], acc[3 * a + 2], v)
                return tuple(acc)

            ts = lax.fori_loop(0, SEG // (L * UNR), step, ts)
        t0, t1, t2 = ts[0], ts[1], ts[2]
        for a in range(1, ACC):
            for v in (ts[3 * a], ts[3 * a + 1], ts[3 * a + 2]):
                t0, t1, t2 = _insert(t0, t1, t2, v)

        def pop(t0, t1, t2):
            m = jnp.max(t0)
            j = plsc.all_reduce_ffs(t0 == m)
            sel = lane == j
            return (m, jnp.where(sel, t1, t0), jnp.where(sel, t2, t1),
                    jnp.where(sel, _NEG, t2))

        m1, t0, t1, t2 = pop(t0, t1, t2)
        m2, t0, t1, t2 = pop(t0, t1, t2)
        m3 = jnp.max(t0)
        res = jnp.where(lane == 0, m1,
                        jnp.where(lane == 1, m2,
                                  jnp.where(lane == 2, m3, jnp.float32(0.0))))
        resv[pl.ds(r * L, L)] = res
    pltpu.sync_copy(resv, out_hbm.at[pl.ds(base * L, RPW * L)])


def kernel(x):
    mesh = plsc.VectorSubcoreMesh(core_axis_name="c", subcore_axis_name="s")
    f = pl.kernel(
        _tec_body,
        mesh=mesh,
        compiler_params=pltpu.CompilerParams(needs_layout_passes=False),
        out_type=jax.ShapeDtypeStruct((R * L,), jnp.float32),
        scratch_types=[
            pltpu.VMEM((RPW * C,), jnp.float32),
            pltpu.VMEM((RPW * L,), jnp.float32),
            pltpu.SemaphoreType.DMA,
            pltpu.SemaphoreType.DMA,
            pltpu.SemaphoreType.DMA,
            pltpu.SemaphoreType.DMA,
        ],
    )
    out = f(x.reshape(R * C))
    return out.reshape(R, L)[:, :3]


# speedup vs baseline: 1.3767x; 1.3767x over previous
import jax
import jax.numpy as jnp
from jax import lax
from jax.experimental import pallas as pl
from jax.experimental.pallas import tpu as pltpu
from jax.experimental.pallas import tpu_sc as plsc

L = 16
R, C = 64, 32768
K = 3
NC, NS = 2, 16
NW = NC * NS
RPW = R // NW
CHUNKS = C // L
ACC = 4
UNR = 8

_NEG = float("-inf")


def _insert(t0, t1, t2, v):
    lo = jnp.minimum(t0, v)
    t0 = jnp.maximum(t0, v)
    lo2 = jnp.minimum(t1, lo)
    t1 = jnp.maximum(t1, lo)
    t2 = jnp.maximum(t2, lo2)
    return t0, t1, t2


def _tec_body(x_hbm, out_hbm, xv, resv, s0, s1):
    cid = lax.axis_index("c")
    sid = lax.axis_index("s")
    wid = sid * NC + cid
    base = wid * RPW
    sems = [s0, s1]
    copies = [
        pltpu.async_copy(x_hbm.at[base + r], xv.at[pl.ds(r * C, C)], sems[r])
        for r in range(RPW)
    ]
    lane = lax.iota(jnp.int32, L)
    full = jnp.full((L,), _NEG, jnp.float32)
    tops = []
    for r in range(RPW):
        copies[r].wait()

        def step(i, carry, _off=r * C):
            acc = list(carry)
            off = _off + i * (L * UNR)
            for j in range(UNR):
                a = j % ACC
                v = xv[pl.ds(off + j * L, L)]
                acc[3 * a], acc[3 * a + 1], acc[3 * a + 2] = _insert(
                    acc[3 * a], acc[3 * a + 1], acc[3 * a + 2], v)
            return tuple(acc)

        ts = lax.fori_loop(0, CHUNKS // UNR, step, (full,) * (3 * ACC))
        t0, t1, t2 = ts[0], ts[1], ts[2]
        for a in range(1, ACC):
            for v in (ts[3 * a], ts[3 * a + 1], ts[3 * a + 2]):
                t0, t1, t2 = _insert(t0, t1, t2, v)

        def pop(t0, t1, t2):
            m = jnp.max(t0)
            j = plsc.all_reduce_ffs(t0 == m)
            sel = lane == j
            return (m, jnp.where(sel, t1, t0), jnp.where(sel, t2, t1),
                    jnp.where(sel, _NEG, t2))

        m1, t0, t1, t2 = pop(t0, t1, t2)
        m2, t0, t1, t2 = pop(t0, t1, t2)
        m3 = jnp.max(t0)
        tops.append((m1, m2, m3))
    res = jnp.zeros((L,), jnp.float32)
    for r in range(RPW):
        for k in range(K):
            res = jnp.where(lane == 8 * r + k, tops[r][k], res)
    resv[...] = res
    for r in range(RPW):
        pltpu.sync_copy(resv.at[pl.ds(8 * r, 8)],
                        out_hbm.at[pl.ds((base + r) * 8, 8)])


def kernel(x):
    mesh = plsc.VectorSubcoreMesh(core_axis_name="c", subcore_axis_name="s")
    f = pl.kernel(
        _tec_body,
        mesh=mesh,
        compiler_params=pltpu.CompilerParams(needs_layout_passes=False),
        out_type=jax.ShapeDtypeStruct((R * 8,), jnp.float32),
        scratch_types=[
            pltpu.VMEM((RPW * C,), jnp.float32),
            pltpu.VMEM((L,), jnp.float32),
            pltpu.SemaphoreType.DMA,
            pltpu.SemaphoreType.DMA,
        ],
    )
    return f(x).reshape(R, 8)[:, :K]


# speedup vs baseline: 1.3797x; 1.0021x over previous
import jax
import jax.numpy as jnp
from jax import lax
from jax.experimental import pallas as pl
from jax.experimental.pallas import tpu as pltpu
from jax.experimental.pallas import tpu_sc as plsc

L = 16
R, C = 64, 32768
K = 3
NC, NS = 2, 16
NW = NC * NS
RPW = R // NW
C2 = C // 2
ACC = 2

_NEG = float("-inf")


def _insert(t0, t1, t2, v):
    lo = jnp.minimum(t0, v)
    t0 = jnp.maximum(t0, v)
    lo2 = jnp.minimum(t1, lo)
    t1 = jnp.maximum(t1, lo)
    t2 = jnp.maximum(t2, lo2)
    return t0, t1, t2


def _tec_body(x_hbm, out_hbm, xv, resv, s00, s01, s10, s11):
    cid = lax.axis_index("c")
    sid = lax.axis_index("s")
    wid = sid * NC + cid
    base = wid * RPW
    sems = [[s00, s01], [s10, s11]]
    copies = [[
        pltpu.async_copy(x_hbm.at[base + r, pl.ds(h * C2, C2)],
                         xv.at[pl.ds(r * C + h * C2, C2)], sems[r][h])
        for h in range(2)] for r in range(RPW)]
    lane = lax.iota(jnp.int32, L)
    full = jnp.full((L,), _NEG, jnp.float32)
    acc = [(full, full, full)] * (RPW * ACC)
    for h in range(2):
        for r in range(RPW):
            copies[r][h].wait()

        def step(i, carry, _h=h):
            acc = list(carry)
            for r in range(RPW):
                for a in range(ACC):
                    off = r * C + _h * C2 + (i * ACC + a) * L
                    v = xv[pl.ds(off, L)]
                    acc[r * ACC + a] = _insert(*acc[r * ACC + a], v)
            return tuple(acc)

        acc = list(lax.fori_loop(0, C2 // (L * ACC), step, tuple(acc)))

    tops = []
    for r in range(RPW):
        t0, t1, t2 = acc[r * ACC]
        for a in range(1, ACC):
            for v in acc[r * ACC + a]:
                t0, t1, t2 = _insert(t0, t1, t2, v)

        def pop(t0, t1, t2):
            m = jnp.max(t0)
            j = plsc.all_reduce_ffs(t0 == m)
            sel = lane == j
            return (m, jnp.where(sel, t1, t0), jnp.where(sel, t2, t1),
                    jnp.where(sel, _NEG, t2))

        m1, t0, t1, t2 = pop(t0, t1, t2)
        m2, t0, t1, t2 = pop(t0, t1, t2)
        m3 = jnp.max(t0)
        tops.append((m1, m2, m3))
    res = jnp.zeros((L,), jnp.float32)
    for r in range(RPW):
        for k in range(K):
            res = jnp.where(lane == 8 * r + k, tops[r][k], res)
    resv[...] = res
    for r in range(RPW):
        pltpu.sync_copy(resv.at[pl.ds(8 * r, 8)],
                        out_hbm.at[pl.ds((base + r) * 8, 8)])


def kernel(x):
    mesh = plsc.VectorSubcoreMesh(core_axis_name="c", subcore_axis_name="s")
    f = pl.kernel(
        _tec_body,
        mesh=mesh,
        compiler_params=pltpu.CompilerParams(needs_layout_passes=False),
        out_type=jax.ShapeDtypeStruct((R * 8,), jnp.float32),
        scratch_types=[
            pltpu.VMEM((RPW * C,), jnp.float32),
            pltpu.VMEM((L,), jnp.float32),
            pltpu.SemaphoreType.DMA,
            pltpu.SemaphoreType.DMA,
            pltpu.SemaphoreType.DMA,
            pltpu.SemaphoreType.DMA,
        ],
    )
    return f(x).reshape(R, 8)[:, :K]


# speedup vs baseline: 1.5558x; 1.1276x over previous
import jax
import jax.numpy as jnp
from jax import lax
from jax.experimental import pallas as pl
from jax.experimental.pallas import tpu as pltpu
from jax.experimental.pallas import tpu_sc as plsc

L = 16
R, C = 64, 32768
K = 3
NC, NS = 2, 16
NW = NC * NS
RSC = NW
RTC = R - RSC
C2 = C // 2
ACC = 4
TCB = 4096

_NEG = float("-inf")


def _insert(t0, t1, t2, v):
    lo = jnp.minimum(t0, v)
    t0 = jnp.maximum(t0, v)
    lo2 = jnp.minimum(t1, lo)
    t1 = jnp.maximum(t1, lo)
    t2 = jnp.maximum(t2, lo2)
    return t0, t1, t2


def _sc_body(x_hbm, out_hbm, xv, resv, s0, s1):
    cid = lax.axis_index("c")
    sid = lax.axis_index("s")
    wid = sid * NC + cid
    sems = [s0, s1]
    copies = [
        pltpu.async_copy(x_hbm.at[wid, pl.ds(h * C2, C2)],
                         xv.at[pl.ds(h * C2, C2)], sems[h])
        for h in range(2)]
    lane = lax.iota(jnp.int32, L)
    full = jnp.full((L,), _NEG, jnp.float32)
    acc = [(full, full, full)] * ACC
    for h in range(2):
        copies[h].wait()

        def step(i, carry, _h=h):
            acc = list(carry)
            for a in range(ACC):
                off = _h * C2 + (i * ACC + a) * L
                acc[a] = _insert(*acc[a], xv[pl.ds(off, L)])
            return tuple(acc)

        acc = list(lax.fori_loop(0, C2 // (L * ACC), step, tuple(acc)))

    t0, t1, t2 = acc[0]
    for a in range(1, ACC):
        for v in acc[a]:
            t0, t1, t2 = _insert(t0, t1, t2, v)

    def pop(t0, t1, t2):
        m = jnp.max(t0)
        j = plsc.all_reduce_ffs(t0 == m)
        sel = lane == j
        return (m, jnp.where(sel, t1, t0), jnp.where(sel, t2, t1),
                jnp.where(sel, _NEG, t2))

    m1, t0, t1, t2 = pop(t0, t1, t2)
    m2, t0, t1, t2 = pop(t0, t1, t2)
    m3 = jnp.max(t0)
    res = jnp.where(lane == 0, m1,
                    jnp.where(lane == 1, m2,
                              jnp.where(lane == 2, m3, jnp.float32(0.0))))
    resv[...] = res
    pltpu.sync_copy(resv.at[pl.ds(0, 8)], out_hbm.at[pl.ds(wid * 8, 8)])


def _tc_body(x_ref, o_ref, t0, t1, t2):
    c = pl.program_id(0)

    @pl.when(c == 0)
    def _init():
        t0[...] = jnp.full((RTC, 128), _NEG, jnp.float32)
        t1[...] = jnp.full((RTC, 128), _NEG, jnp.float32)
        t2[...] = jnp.full((RTC, 128), _NEG, jnp.float32)

    blk = x_ref[...]
    a0, a1, a2 = t0[...], t1[...], t2[...]
    for j in range(TCB // 128):
        a0, a1, a2 = _insert(a0, a1, a2, blk[:, 128 * j:128 * (j + 1)])
    t0[...], t1[...], t2[...] = a0, a1, a2

    @pl.when(c == C // TCB - 1)
    def _fin():
        a0, a1, a2 = t0[...], t1[...], t2[...]
        iota = lax.broadcasted_iota(jnp.int32, (RTC, 128), 1)

        def pop(a0, a1, a2):
            m = jnp.max(a0, axis=1, keepdims=True)
            j = jnp.min(jnp.where(a0 == m, iota, 128), axis=1, keepdims=True)
            sel = iota == j
            return (m, jnp.where(sel, a1, a0), jnp.where(sel, a2, a1),
                    jnp.where(sel, _NEG, a2))

        m1, a0, a1, a2 = pop(a0, a1, a2)
        m2, a0, a1, a2 = pop(a0, a1, a2)
        m3 = jnp.max(a0, axis=1, keepdims=True)
        o_ref[...] = jnp.where(iota == 0, m1,
                               jnp.where(iota == 1, m2,
                                         jnp.where(iota == 2, m3,
                                                   jnp.float32(0.0))))


def kernel(x):
    mesh = plsc.VectorSubcoreMesh(core_axis_name="c", subcore_axis_name="s")
    f_sc = pl.kernel(
        _sc_body,
        mesh=mesh,
        compiler_params=pltpu.CompilerParams(needs_layout_passes=False),
        out_type=jax.ShapeDtypeStruct((RSC * 8,), jnp.float32),
        scratch_types=[
            pltpu.VMEM((C,), jnp.float32),
            pltpu.VMEM((L,), jnp.float32),
            pltpu.SemaphoreType.DMA,
            pltpu.SemaphoreType.DMA,
        ],
    )
    f_tc = pl.pallas_call(
        _tc_body,
        grid=(C // TCB,),
        in_specs=[pl.BlockSpec((RTC, TCB), lambda c: (1, c))],
        out_specs=pl.BlockSpec((RTC, 128), lambda c: (0, 0)),
        out_shape=jax.ShapeDtypeStruct((RTC, 128), jnp.float32),
        scratch_shapes=[pltpu.VMEM((RTC, 128), jnp.float32)] * 3,
    )
    sc_out = f_sc(x).reshape(RSC, 8)[:, :K]
    tc_out = f_tc(x)[:, :K]
    return jnp.concatenate([sc_out, tc_out], axis=0)


# speedup vs baseline: 1.5762x; 1.0131x over previous
import jax
import jax.numpy as jnp
from jax import lax
from jax.experimental import pallas as pl
from jax.experimental.pallas import tpu as pltpu
from jax.experimental.pallas import tpu_sc as plsc

L = 16
R, C = 64, 32768
K = 3
NC, NS = 2, 16
NW = NC * NS
RSC = NW
RTC = R - RSC
C2 = C // 2
ACC = 4
TCB = 8192

_NEG = float("-inf")


def _insert(t0, t1, t2, v):
    lo = jnp.minimum(t0, v)
    t0 = jnp.maximum(t0, v)
    lo2 = jnp.minimum(t1, lo)
    t1 = jnp.maximum(t1, lo)
    t2 = jnp.maximum(t2, lo2)
    return t0, t1, t2


def _sc_body(x_hbm, out_hbm, xv, resv, s0, s1):
    cid = lax.axis_index("c")
    sid = lax.axis_index("s")
    wid = sid * NC + cid
    sems = [s0, s1]
    copies = [
        pltpu.async_copy(x_hbm.at[wid, pl.ds(h * C2, C2)],
                         xv.at[pl.ds(h * C2, C2)], sems[h])
        for h in range(2)]
    lane = lax.iota(jnp.int32, L)
    full = jnp.full((L,), _NEG, jnp.float32)
    acc = [(full, full, full)] * ACC
    for h in range(2):
        copies[h].wait()

        def step(i, carry, _h=h):
            acc = list(carry)
            for a in range(ACC):
                off = _h * C2 + (i * ACC + a) * L
                acc[a] = _insert(*acc[a], xv[pl.ds(off, L)])
            return tuple(acc)

        acc = list(lax.fori_loop(0, C2 // (L * ACC), step, tuple(acc)))

    t0, t1, t2 = acc[0]
    for a in range(1, ACC):
        for v in acc[a]:
            t0, t1, t2 = _insert(t0, t1, t2, v)

    def pop(t0, t1, t2):
        m = jnp.max(t0)
        j = plsc.all_reduce_ffs(t0 == m)
        sel = lane == j
        return (m, jnp.where(sel, t1, t0), jnp.where(sel, t2, t1),
                jnp.where(sel, _NEG, t2))

    m1, t0, t1, t2 = pop(t0, t1, t2)
    m2, t0, t1, t2 = pop(t0, t1, t2)
    m3 = jnp.max(t0)
    res = jnp.where(lane == 0, m1,
                    jnp.where(lane == 1, m2,
                              jnp.where(lane == 2, m3, jnp.float32(0.0))))
    resv[...] = res
    pltpu.sync_copy(resv.at[pl.ds(0, 8)], out_hbm.at[pl.ds(wid * 8, 8)])


def _tc_body(x_ref, o_ref, t0, t1, t2):
    c = pl.program_id(0)

    @pl.when(c == 0)
    def _init():
        t0[...] = jnp.full((RTC, 128), _NEG, jnp.float32)
        t1[...] = jnp.full((RTC, 128), _NEG, jnp.float32)
        t2[...] = jnp.full((RTC, 128), _NEG, jnp.float32)

    blk = x_ref[...]
    a0, a1, a2 = t0[...], t1[...], t2[...]
    for j in range(TCB // 128):
        a0, a1, a2 = _insert(a0, a1, a2, blk[:, 128 * j:128 * (j + 1)])
    t0[...], t1[...], t2[...] = a0, a1, a2

    @pl.when(c == C // TCB - 1)
    def _fin():
        a0, a1, a2 = t0[...], t1[...], t2[...]
        iota = lax.broadcasted_iota(jnp.int32, (RTC, 128), 1)

        def pop(a0, a1, a2):
            m = jnp.max(a0, axis=1, keepdims=True)
            j = jnp.min(jnp.where(a0 == m, iota, 128), axis=1, keepdims=True)
            sel = iota == j
            return (m, jnp.where(sel, a1, a0), jnp.where(sel, a2, a1),
                    jnp.where(sel, _NEG, a2))

        m1, a0, a1, a2 = pop(a0, a1, a2)
        m2, a0, a1, a2 = pop(a0, a1, a2)
        m3 = jnp.max(a0, axis=1, keepdims=True)
        res = jnp.where(iota == 0, m1,
                        jnp.where(iota == 1, m2,
                                  jnp.where(iota == 2, m3, jnp.float32(0.0))))
        o_ref[...] = res[:, :8]


def kernel(x):
    mesh = plsc.VectorSubcoreMesh(core_axis_name="c", subcore_axis_name="s")
    f_sc = pl.kernel(
        _sc_body,
        mesh=mesh,
        compiler_params=pltpu.CompilerParams(needs_layout_passes=False),
        out_type=jax.ShapeDtypeStruct((RSC * 8,), jnp.float32),
        scratch_types=[
            pltpu.VMEM((C,), jnp.float32),
            pltpu.VMEM((L,), jnp.float32),
            pltpu.SemaphoreType.DMA,
            pltpu.SemaphoreType.DMA,
        ],
    )
    f_tc = pl.pallas_call(
        _tc_body,
        grid=(C // TCB,),
        in_specs=[pl.BlockSpec((RTC, TCB), lambda c: (1, c))],
        out_specs=pl.BlockSpec((RTC, 8), lambda c: (0, 0)),
        out_shape=jax.ShapeDtypeStruct((RTC, 8), jnp.float32),
        scratch_shapes=[pltpu.VMEM((RTC, 128), jnp.float32)] * 3,
    )
    out8 = jnp.concatenate([f_sc(x).reshape(RSC, 8), f_tc(x)], axis=0)
    return out8[:, :K]


# speedup vs baseline: 1.5763x; 1.0001x over previous
import jax
import jax.numpy as jnp
from jax import lax
from jax.experimental import pallas as pl
from jax.experimental.pallas import tpu as pltpu
from jax.experimental.pallas import tpu_sc as plsc

L = 16
R, C = 64, 32768
K = 3
NC, NS = 2, 16
NW = NC * NS
RSC = NW
RTC = R - RSC
C2 = C // 2
ACC = 4
TCB = 8192

_NEG = float("-inf")


def _insert(t0, t1, t2, v):
    lo = jnp.minimum(t0, v)
    t0 = jnp.maximum(t0, v)
    lo2 = jnp.minimum(t1, lo)
    t1 = jnp.maximum(t1, lo)
    t2 = jnp.maximum(t2, lo2)
    return t0, t1, t2


def _sc_body(x_hbm, out_hbm, xv, resv, s0, s1, s2, s3):
    cid = lax.axis_index("c")
    sid = lax.axis_index("s")
    wid = sid * NC + cid
    sems = [s0, s1, s2, s3]
    nseg = len(sems)
    seg = C // nseg
    copies = [
        pltpu.async_copy(x_hbm.at[wid, pl.ds(h * seg, seg)],
                         xv.at[pl.ds(h * seg, seg)], sems[h])
        for h in range(nseg)]
    lane = lax.iota(jnp.int32, L)
    full = jnp.full((L,), _NEG, jnp.float32)
    acc = [(full, full, full)] * ACC
    for h in range(nseg):
        copies[h].wait()

        def step(i, carry, _h=h):
            acc = list(carry)
            for a in range(ACC):
                off = _h * seg + (i * ACC + a) * L
                acc[a] = _insert(*acc[a], xv[pl.ds(off, L)])
            return tuple(acc)

        acc = list(lax.fori_loop(0, seg // (L * ACC), step, tuple(acc)))

    t0, t1, t2 = acc[0]
    for a in range(1, ACC):
        for v in acc[a]:
            t0, t1, t2 = _insert(t0, t1, t2, v)

    def pop(t0, t1, t2):
        m = jnp.max(t0)
        j = plsc.all_reduce_ffs(t0 == m)
        sel = lane == j
        return (m, jnp.where(sel, t1, t0), jnp.where(sel, t2, t1),
                jnp.where(sel, _NEG, t2))

    m1, t0, t1, t2 = pop(t0, t1, t2)
    m2, t0, t1, t2 = pop(t0, t1, t2)
    m3 = jnp.max(t0)
    res = jnp.where(lane == 0, m1,
                    jnp.where(lane == 1, m2,
                              jnp.where(lane == 2, m3, jnp.float32(0.0))))
    resv[...] = res
    pltpu.sync_copy(resv.at[pl.ds(0, 8)], out_hbm.at[pl.ds(wid * 8, 8)])


def _tc_body(x_ref, o_ref, t0, t1, t2):
    c = pl.program_id(0)

    @pl.when(c == 0)
    def _init():
        t0[...] = jnp.full((RTC, 128), _NEG, jnp.float32)
        t1[...] = jnp.full((RTC, 128), _NEG, jnp.float32)
        t2[...] = jnp.full((RTC, 128), _NEG, jnp.float32)

    blk = x_ref[...]
    a0, a1, a2 = t0[...], t1[...], t2[...]
    for j in range(TCB // 128):
        a0, a1, a2 = _insert(a0, a1, a2, blk[:, 128 * j:128 * (j + 1)])
    t0[...], t1[...], t2[...] = a0, a1, a2

    @pl.when(c == C // TCB - 1)
    def _fin():
        a0, a1, a2 = t0[...], t1[...], t2[...]
        iota = lax.broadcasted_iota(jnp.int32, (RTC, 128), 1)

        def pop(a0, a1, a2):
            m = jnp.max(a0, axis=1, keepdims=True)
            j = jnp.min(jnp.where(a0 == m, iota, 128), axis=1, keepdims=True)
            sel = iota == j
            return (m, jnp.where(sel, a1, a0), jnp.where(sel, a2, a1),
                    jnp.where(sel, _NEG, a2))

        m1, a0, a1, a2 = pop(a0, a1, a2)
        m2, a0, a1, a2 = pop(a0, a1, a2)
        m3 = jnp.max(a0, axis=1, keepdims=True)
        res = jnp.where(iota == 0, m1,
                        jnp.where(iota == 1, m2,
                                  jnp.where(iota == 2, m3, jnp.float32(0.0))))
        o_ref[...] = res[:, :8]


def kernel(x):
    mesh = plsc.VectorSubcoreMesh(core_axis_name="c", subcore_axis_name="s")
    f_sc = pl.kernel(
        _sc_body,
        mesh=mesh,
        compiler_params=pltpu.CompilerParams(needs_layout_passes=False),
        out_type=jax.ShapeDtypeStruct((RSC * 8,), jnp.float32),
        scratch_types=[
            pltpu.VMEM((C,), jnp.float32),
            pltpu.VMEM((L,), jnp.float32),
            pltpu.SemaphoreType.DMA,
            pltpu.SemaphoreType.DMA,
            pltpu.SemaphoreType.DMA,
            pltpu.SemaphoreType.DMA,
        ],
    )
    f_tc = pl.pallas_call(
        _tc_body,
        grid=(C // TCB,),
        in_specs=[pl.BlockSpec((RTC, TCB), lambda c: (1, c))],
        out_specs=pl.BlockSpec((RTC, 8), lambda c: (0, 0)),
        out_shape=jax.ShapeDtypeStruct((RTC, 8), jnp.float32),
        scratch_shapes=[pltpu.VMEM((RTC, 128), jnp.float32)] * 3,
    )
    out8 = jnp.concatenate([f_sc(x).reshape(RSC, 8), f_tc(x)], axis=0)
    return out8[:, :K]
